# Initial kernel scaffold; baseline (speedup 1.0000x reference)
import jax
import jax.numpy as jnp
from jax.experimental import pallas as pl


def kernel(x, edge_attr, key_unused, params):
    raise NotImplementedError("write your pallas kernel here")


# speedup vs baseline: 61.6429x; 61.6429x over previous
import jax
import jax.numpy as jnp
from jax.experimental import pallas as pl
from jax.experimental.pallas import tpu as pltpu

_B, _N, _DIN, _H, _OUT = 4, 128, 128, 128, 128
_NL, _DE = 4, 4
_TI = 32
_T = _N // _TI
_EN = _TI * _N

_F32 = jnp.float32
_BF = jnp.bfloat16


def _gnn_body(x_ref, ea_ref,
              w1i_ref, b1i_ref, w2i_ref, b2i_ref,
              we1a_ref, we1b_ref, we1e_ref, be1_ref,
              we2_ref, be2_ref,
              wn1a_ref, wn1b_ref, bn1_ref, wn2_ref, bn2_ref,
              h_ref,
              e_scr, a_scr, c_scr, mi_scr):
    l = pl.program_id(1)
    t = pl.program_id(2)

    @pl.when((l == 0) & (t == 0))
    def _emb_in():
        xb = x_ref[0].astype(_BF)
        h1 = jnp.maximum(
            jnp.dot(xb, w1i_ref[...], preferred_element_type=_F32) + b1i_ref[...], 0.0)
        h_ref[0] = (jnp.dot(h1.astype(_BF), w2i_ref[...], preferred_element_type=_F32)
                    + b2i_ref[...])

    @pl.when(l == 0)
    def _edge_embed():
        e_all = jnp.dot(ea_ref[0], we1e_ref[...], preferred_element_type=_F32)
        for li in range(_NL):
            e_scr[li, t] = e_all[:, li * _H:(li + 1) * _H]

    @pl.when(t == 0)
    def _layer_head():
        hb = h_ref[0].astype(_BF)
        a_scr[...] = (jnp.dot(hb, we1a_ref[l], preferred_element_type=_F32)
                      + be1_ref[l])
        c_scr[...] = jnp.dot(hb, we1b_ref[l], preferred_element_type=_F32)

    pre = (e_scr[l, t].reshape(_TI, _N, _H)
           + a_scr[pl.ds(t * _TI, _TI), :].reshape(_TI, 1, _H)
           + c_scr[...].reshape(1, _N, _H))
    m1 = jnp.maximum(pre, 0.0).reshape(_EN, _H).astype(_BF)
    m2 = jnp.maximum(
        jnp.dot(m1, we2_ref[l], preferred_element_type=_F32) + be2_ref[l], 0.0)
    mi_scr[pl.ds(t * _TI, _TI), :] = m2.reshape(_TI, _N, _H).sum(axis=1)

    @pl.when(t == _T - 1)
    def _node_op():
        h = h_ref[0]
        u = jnp.maximum(
            jnp.dot(h.astype(_BF), wn1a_ref[l], preferred_element_type=_F32)
            + jnp.dot(mi_scr[...].astype(_BF), wn1b_ref[l], preferred_element_type=_F32)
            + bn1_ref[l], 0.0)
        h_ref[0] = (h + jnp.dot(u.astype(_BF), wn2_ref[l], preferred_element_type=_F32)
                    + bn2_ref[l])


def _out_body(hf_ref, w1_ref, b1_ref, w2_ref, b2_ref, o_ref):
    h1 = jnp.maximum(
        jnp.dot(hf_ref[...], w1_ref[...], preferred_element_type=_F32) + b1_ref[...], 0.0)
    o_ref[...] = (jnp.dot(h1.astype(_BF), w2_ref[...], preferred_element_type=_F32)
                  + b2_ref[...])


def kernel(x, edge_attr, key_unused, params):
    p = params
    w1i = p['emb_in_W1'].astype(_BF)
    b1i = p['emb_in_b1'].reshape(1, _H)
    w2i = p['emb_in_W2'].astype(_BF)
    b2i = p['emb_in_b2'].reshape(1, _H)
    we1a = jnp.stack([p['l%d_We1' % l][:_H] for l in range(_NL)]).astype(_BF)
    we1b = jnp.stack([p['l%d_We1' % l][_H:2 * _H] for l in range(_NL)]).astype(_BF)
    we1e = jnp.concatenate(
        [p['l%d_We1' % l][2 * _H:] for l in range(_NL)], axis=1).astype(_BF)
    be1 = jnp.stack([p['l%d_be1' % l].reshape(1, _H) for l in range(_NL)])
    we2 = jnp.stack([p['l%d_We2' % l] for l in range(_NL)]).astype(_BF)
    be2 = jnp.stack([p['l%d_be2' % l].reshape(1, _H) for l in range(_NL)])
    wn1a = jnp.stack([p['l%d_Wn1' % l][:_H] for l in range(_NL)]).astype(_BF)
    wn1b = jnp.stack([p['l%d_Wn1' % l][_H:] for l in range(_NL)]).astype(_BF)
    bn1 = jnp.stack([p['l%d_bn1' % l].reshape(1, _H) for l in range(_NL)])
    wn2 = jnp.stack([p['l%d_Wn2' % l] for l in range(_NL)]).astype(_BF)
    bn2 = jnp.stack([p['l%d_bn2' % l].reshape(1, _H) for l in range(_NL)])
    eab = edge_attr.astype(_BF)

    _c2 = lambda b, l, t: (0, 0)
    _c3 = lambda b, l, t: (0, 0, 0)
    h = pl.pallas_call(
        _gnn_body,
        grid=(_B, _NL, _T),
        in_specs=[
            pl.BlockSpec((1, _N, _DIN), lambda b, l, t: (b, 0, 0)),
            pl.BlockSpec((1, _EN, _DE), lambda b, l, t: (b, t, 0)),
            pl.BlockSpec((_DIN, _H), _c2),
            pl.BlockSpec((1, _H), _c2),
            pl.BlockSpec((_H, _H), _c2),
            pl.BlockSpec((1, _H), _c2),
            pl.BlockSpec((_NL, _H, _H), _c3),
            pl.BlockSpec((_NL, _H, _H), _c3),
            pl.BlockSpec((_DE, _NL * _H), _c2),
            pl.BlockSpec((_NL, 1, _H), _c3),
            pl.BlockSpec((_NL, _H, _H), _c3),
            pl.BlockSpec((_NL, 1, _H), _c3),
            pl.BlockSpec((_NL, _H, _H), _c3),
            pl.BlockSpec((_NL, _H, _H), _c3),
            pl.BlockSpec((_NL, 1, _H), _c3),
            pl.BlockSpec((_NL, _H, _H), _c3),
            pl.BlockSpec((_NL, 1, _H), _c3),
        ],
        out_specs=pl.BlockSpec((1, _N, _H), lambda b, l, t: (b, 0, 0)),
        out_shape=jax.ShapeDtypeStruct((_B, _N, _H), _F32),
        scratch_shapes=[
            pltpu.VMEM((_NL, _T, _EN, _H), _F32),
            pltpu.VMEM((_N, _H), _F32),
            pltpu.VMEM((_N, _H), _F32),
            pltpu.VMEM((_N, _H), _F32),
        ],
        compiler_params=pltpu.CompilerParams(
            dimension_semantics=("arbitrary", "arbitrary", "arbitrary")),
    )(x, eab, w1i, b1i, w2i, b2i,
      we1a, we1b, we1e, be1, we2, be2, wn1a, wn1b, bn1, wn2, bn2)

    hf = h.reshape(_B, _N * _H).astype(_BF)
    out = pl.pallas_call(
        _out_body,
        out_shape=jax.ShapeDtypeStruct((_B, _OUT), _F32),
    )(hf, p['emb_out_W1'].astype(_BF), p['emb_out_b1'].reshape(1, _H),
      p['emb_out_W2'].astype(_BF), p['emb_out_b2'].reshape(1, _OUT))
    return out


# speedup vs baseline: 62.1380x; 1.0080x over previous
import jax
import jax.numpy as jnp
from jax.experimental import pallas as pl
from jax.experimental.pallas import tpu as pltpu

_B, _N, _DIN, _H, _OUT = 4, 128, 128, 128, 128
_NL, _DE = 4, 4
_TI = 32
_T = _N // _TI
_NP = _N // 2
_EP = _TI * _NP
_HP = 2 * _H

_F32 = jnp.float32
_BF = jnp.bfloat16


def _gnn_body(x_ref, ea_ref,
              w1i_ref, b1i_ref, w2i_ref, b2i_ref,
              we1a_ref, we1b_ref, we1e_ref, be1_ref,
              we2_ref, be2_ref,
              wn1a_ref, wn1b_ref, bn1_ref, wn2_ref, bn2_ref,
              h_ref,
              e_scr, a_scr, c_scr, mi_scr):
    l = pl.program_id(1)
    t = pl.program_id(2)

    @pl.when((l == 0) & (t == 0))
    def _emb_in():
        xb = x_ref[0].astype(_BF)
        h1 = jnp.maximum(
            jnp.dot(xb, w1i_ref[...], preferred_element_type=_F32) + b1i_ref[...], 0.0)
        h_ref[0] = (jnp.dot(h1.astype(_BF), w2i_ref[...], preferred_element_type=_F32)
                    + b2i_ref[...])

    @pl.when(l == 0)
    def _edge_embed():
        e_all = jnp.dot(ea_ref[0, 0], we1e_ref[...], preferred_element_type=_F32)
        for li in range(_NL):
            e_scr[li, t] = e_all[:, li * _HP:(li + 1) * _HP].astype(_BF)

    @pl.when(t == 0)
    def _layer_head():
        hb = h_ref[0].astype(_BF)
        a = jnp.dot(hb, we1a_ref[l], preferred_element_type=_F32) + be1_ref[l]
        a_scr[...] = jnp.concatenate([a, a], axis=1)
        c = jnp.dot(hb, we1b_ref[l], preferred_element_type=_F32)
        c_scr[...] = jnp.concatenate([c[:_NP], c[_NP:]], axis=1)

    pre = (e_scr[l, t].astype(_F32).reshape(_TI, _NP, _HP)
           + a_scr[pl.ds(t * _TI, _TI), :].reshape(_TI, 1, _HP)
           + c_scr[...].reshape(1, _NP, _HP))
    m1 = jnp.maximum(pre, 0.0).reshape(_EP, _HP).astype(_BF)
    m2 = jnp.maximum(
        jnp.dot(m1, we2_ref[l], preferred_element_type=_F32) + be2_ref[l], 0.0)
    ms = m2.reshape(_TI, _NP, _HP).sum(axis=1)
    mi_scr[pl.ds(t * _TI, _TI), :] = ms[:, :_H] + ms[:, _H:]

    @pl.when(t == _T - 1)
    def _node_op():
        h = h_ref[0]
        u = jnp.maximum(
            jnp.dot(h.astype(_BF), wn1a_ref[l], preferred_element_type=_F32)
            + jnp.dot(mi_scr[...].astype(_BF), wn1b_ref[l], preferred_element_type=_F32)
            + bn1_ref[l], 0.0)
        h_ref[0] = (h + jnp.dot(u.astype(_BF), wn2_ref[l], preferred_element_type=_F32)
                    + bn2_ref[l])


def _out_body(hf_ref, w1_ref, b1_ref, w2_ref, b2_ref, o_ref):
    h1 = jnp.maximum(
        jnp.dot(hf_ref[...], w1_ref[...], preferred_element_type=_F32) + b1_ref[...], 0.0)
    o_ref[...] = (jnp.dot(h1.astype(_BF), w2_ref[...], preferred_element_type=_F32)
                  + b2_ref[...])


def _blockdiag2(w):
    k, n = w.shape
    return jnp.zeros((2 * k, 2 * n), w.dtype).at[:k, :n].set(w).at[k:, n:].set(w)


def kernel(x, edge_attr, key_unused, params):
    p = params
    w1i = p['emb_in_W1'].astype(_BF)
    b1i = p['emb_in_b1'].reshape(1, _H)
    w2i = p['emb_in_W2'].astype(_BF)
    b2i = p['emb_in_b2'].reshape(1, _H)
    we1a = jnp.stack([p['l%d_We1' % l][:_H] for l in range(_NL)]).astype(_BF)
    we1b = jnp.stack([p['l%d_We1' % l][_H:2 * _H] for l in range(_NL)]).astype(_BF)
    we1e = jnp.concatenate(
        [_blockdiag2(p['l%d_We1' % l][2 * _H:]) for l in range(_NL)],
        axis=1).astype(_BF)
    be1 = jnp.stack([p['l%d_be1' % l].reshape(1, _H) for l in range(_NL)])
    we2 = jnp.stack([_blockdiag2(p['l%d_We2' % l]) for l in range(_NL)]).astype(_BF)
    be2 = jnp.stack(
        [jnp.tile(p['l%d_be2' % l].reshape(1, _H), (1, 2)) for l in range(_NL)])
    wn1a = jnp.stack([p['l%d_Wn1' % l][:_H] for l in range(_NL)]).astype(_BF)
    wn1b = jnp.stack([p['l%d_Wn1' % l][_H:] for l in range(_NL)]).astype(_BF)
    bn1 = jnp.stack([p['l%d_bn1' % l].reshape(1, _H) for l in range(_NL)])
    wn2 = jnp.stack([p['l%d_Wn2' % l] for l in range(_NL)]).astype(_BF)
    bn2 = jnp.stack([p['l%d_bn2' % l].reshape(1, _H) for l in range(_NL)])
    eap = (edge_attr.astype(_BF)
           .reshape(_B, _N, 2, _NP, _DE)
           .transpose(0, 1, 3, 2, 4)
           .reshape(_B, _T, _EP, 2 * _DE))

    _c2 = lambda b, l, t: (0, 0)
    _c3 = lambda b, l, t: (0, 0, 0)
    h = pl.pallas_call(
        _gnn_body,
        grid=(_B, _NL, _T),
        in_specs=[
            pl.BlockSpec((1, _N, _DIN), lambda b, l, t: (b, 0, 0)),
            pl.BlockSpec((1, 1, _EP, 2 * _DE), lambda b, l, t: (b, t, 0, 0)),
            pl.BlockSpec((_DIN, _H), _c2),
            pl.BlockSpec((1, _H), _c2),
            pl.BlockSpec((_H, _H), _c2),
            pl.BlockSpec((1, _H), _c2),
            pl.BlockSpec((_NL, _H, _H), _c3),
            pl.BlockSpec((_NL, _H, _H), _c3),
            pl.BlockSpec((2 * _DE, _NL * _HP), _c2),
            pl.BlockSpec((_NL, 1, _H), _c3),
            pl.BlockSpec((_NL, _HP, _HP), _c3),
            pl.BlockSpec((_NL, 1, _HP), _c3),
            pl.BlockSpec((_NL, _H, _H), _c3),
            pl.BlockSpec((_NL, _H, _H), _c3),
            pl.BlockSpec((_NL, 1, _H), _c3),
            pl.BlockSpec((_NL, _H, _H), _c3),
            pl.BlockSpec((_NL, 1, _H), _c3),
        ],
        out_specs=pl.BlockSpec((1, _N, _H), lambda b, l, t: (b, 0, 0)),
        out_shape=jax.ShapeDtypeStruct((_B, _N, _H), _F32),
        scratch_shapes=[
            pltpu.VMEM((_NL, _T, _EP, _HP), _BF),
            pltpu.VMEM((_N, _HP), _F32),
            pltpu.VMEM((_NP, _HP), _F32),
            pltpu.VMEM((_N, _H), _F32),
        ],
        compiler_params=pltpu.CompilerParams(
            dimension_semantics=("arbitrary", "arbitrary", "arbitrary")),
    )(x, eap, w1i, b1i, w2i, b2i,
      we1a, we1b, we1e, be1, we2, be2, wn1a, wn1b, bn1, wn2, bn2)

    hf = h.reshape(_B, _N * _H).astype(_BF)
    out = pl.pallas_call(
        _out_body,
        out_shape=jax.ShapeDtypeStruct((_B, _OUT), _F32),
    )(hf, p['emb_out_W1'].astype(_BF), p['emb_out_b1'].reshape(1, _H),
      p['emb_out_W2'].astype(_BF), p['emb_out_b2'].reshape(1, _OUT))
    return out


# speedup vs baseline: 88.7098x; 1.4276x over previous
import jax
import jax.numpy as jnp
from jax.experimental import pallas as pl
from jax.experimental.pallas import tpu as pltpu

_B, _N, _DIN, _H, _OUT = 4, 128, 128, 128, 128
_NL, _DE = 4, 4
_NP = _N // 2
_EP = _N * _NP
_HP = 2 * _H

_F32 = jnp.float32
_BF = jnp.bfloat16


def _gnn_body(x_ref, ea_ref, wi_ref, *wl_refs):
    h_ref = wl_refs[-1]
    w1i_ref, b1i_ref, w2i_ref, b2i_ref = wi_ref

    xb = x_ref[0].astype(_BF)
    h1 = jnp.maximum(
        jnp.dot(xb, w1i_ref[...], preferred_element_type=_F32) + b1i_ref[...], 0.0)
    h = (jnp.dot(h1.astype(_BF), w2i_ref[...], preferred_element_type=_F32)
         + b2i_ref[...])

    ea = ea_ref[0]
    for l in range(_NL):
        (we1a_ref, we1b_ref, we1e_ref, be1_ref, we2_ref, be2_ref,
         wn1a_ref, wn1b_ref, bn1_ref, wn2_ref, bn2_ref) = wl_refs[11 * l:11 * (l + 1)]
        hb = h.astype(_BF)
        a = jnp.dot(hb, we1a_ref[...], preferred_element_type=_F32) + be1_ref[...]
        ap = jnp.concatenate([a, a], axis=1)
        c = jnp.dot(hb, we1b_ref[...], preferred_element_type=_F32)
        cp = jnp.concatenate([c[:_NP], c[_NP:]], axis=1)
        e = jnp.dot(ea, we1e_ref[...], preferred_element_type=_F32)
        pre = (e.reshape(_N, _NP, _HP)
               + ap.reshape(_N, 1, _HP)
               + cp.reshape(1, _NP, _HP))
        m1 = jnp.maximum(pre, 0.0).reshape(_EP, _HP).astype(_BF)
        m2 = jnp.maximum(
            jnp.dot(m1, we2_ref[...], preferred_element_type=_F32) + be2_ref[...], 0.0)
        ms = m2.reshape(_N, _NP, _HP).sum(axis=1)
        mi = ms[:, :_H] + ms[:, _H:]
        u = jnp.maximum(
            jnp.dot(hb, wn1a_ref[...], preferred_element_type=_F32)
            + jnp.dot(mi.astype(_BF), wn1b_ref[...], preferred_element_type=_F32)
            + bn1_ref[...], 0.0)
        h = (h + jnp.dot(u.astype(_BF), wn2_ref[...], preferred_element_type=_F32)
             + bn2_ref[...])

    h_ref[0] = h


def _out_body(hf_ref, w1_ref, b1_ref, w2_ref, b2_ref, o_ref):
    h1 = jnp.maximum(
        jnp.dot(hf_ref[...], w1_ref[...], preferred_element_type=_F32) + b1_ref[...], 0.0)
    o_ref[...] = (jnp.dot(h1.astype(_BF), w2_ref[...], preferred_element_type=_F32)
                  + b2_ref[...])


def _blockdiag2(w):
    k, n = w.shape
    return jnp.zeros((2 * k, 2 * n), w.dtype).at[:k, :n].set(w).at[k:, n:].set(w)


def kernel(x, edge_attr, key_unused, params):
    p = params
    wi = (p['emb_in_W1'].astype(_BF), p['emb_in_b1'].reshape(1, _H),
          p['emb_in_W2'].astype(_BF), p['emb_in_b2'].reshape(1, _H))
    wl = []
    for l in range(_NL):
        we1 = p['l%d_We1' % l]
        wn1 = p['l%d_Wn1' % l]
        wl += [
            we1[:_H].astype(_BF),
            we1[_H:2 * _H].astype(_BF),
            _blockdiag2(we1[2 * _H:]).astype(_BF),
            p['l%d_be1' % l].reshape(1, _H),
            _blockdiag2(p['l%d_We2' % l]).astype(_BF),
            jnp.tile(p['l%d_be2' % l].reshape(1, _H), (1, 2)),
            wn1[:_H].astype(_BF),
            wn1[_H:].astype(_BF),
            p['l%d_bn1' % l].reshape(1, _H),
            p['l%d_Wn2' % l].astype(_BF),
            p['l%d_bn2' % l].reshape(1, _H),
        ]
    eap = (edge_attr.astype(_BF)
           .reshape(_B, _N, 2, _NP, _DE)
           .transpose(0, 1, 3, 2, 4)
           .reshape(_B, _EP, 2 * _DE))

    _cw = lambda b: tuple(0 for _ in range(2))
    h = pl.pallas_call(
        _gnn_body,
        grid=(_B,),
        in_specs=[
            pl.BlockSpec((1, _N, _DIN), lambda b: (b, 0, 0)),
            pl.BlockSpec((1, _EP, 2 * _DE), lambda b: (b, 0, 0)),
            tuple(pl.BlockSpec(w.shape, _cw) for w in wi),
        ] + [pl.BlockSpec(w.shape, _cw) for w in wl],
        out_specs=pl.BlockSpec((1, _N, _H), lambda b: (b, 0, 0)),
        out_shape=jax.ShapeDtypeStruct((_B, _N, _H), _F32),
        compiler_params=pltpu.CompilerParams(
            dimension_semantics=("arbitrary",)),
    )(x, eap, wi, *wl)

    hf = h.reshape(_B, _N * _H).astype(_BF)
    out = pl.pallas_call(
        _out_body,
        out_shape=jax.ShapeDtypeStruct((_B, _OUT), _F32),
    )(hf, p['emb_out_W1'].astype(_BF), p['emb_out_b1'].reshape(1, _H),
      p['emb_out_W2'].astype(_BF), p['emb_out_b2'].reshape(1, _OUT))
    return out


# speedup vs baseline: 107.9425x; 1.2168x over previous
import jax
import jax.numpy as jnp
from jax.experimental import pallas as pl
from jax.experimental.pallas import tpu as pltpu

_B, _N, _DIN, _H, _OUT = 4, 128, 128, 128, 128
_NL, _DE = 4, 4
_NP = _N // 2
_EP = _N * _NP
_HP = 2 * _H

_F32 = jnp.float32
_BF = jnp.bfloat16


def _bd2(w):
    z = jnp.zeros(w.shape, _BF)
    wb = w.astype(_BF)
    return jnp.concatenate(
        [jnp.concatenate([wb, z], axis=1), jnp.concatenate([z, wb], axis=1)], axis=0)


def _gnn_body(x_ref, ea_ref, w1i_ref, b1i_ref, w2i_ref, b2i_ref, *rest):
    h_ref = rest[-1]
    wl_refs = rest[:-1]

    xb = x_ref[0].astype(_BF)
    h1 = jnp.maximum(
        jnp.dot(xb, w1i_ref[...].astype(_BF), preferred_element_type=_F32)
        + b1i_ref[...], 0.0)
    h = (jnp.dot(h1.astype(_BF), w2i_ref[...].astype(_BF), preferred_element_type=_F32)
         + b2i_ref[...])

    ea = ea_ref[0]
    for l in range(_NL):
        (we1_ref, be1_ref, we2_ref, be2_ref,
         wn1_ref, bn1_ref, wn2_ref, bn2_ref) = wl_refs[8 * l:8 * (l + 1)]
        we1 = we1_ref[...]
        hb = h.astype(_BF)
        a = (jnp.dot(hb, we1[:_H].astype(_BF), preferred_element_type=_F32)
             + be1_ref[...])
        ap = jnp.concatenate([a, a], axis=1)
        c = jnp.dot(hb, we1[_H:2 * _H].astype(_BF), preferred_element_type=_F32)
        cp = jnp.concatenate([c[:_NP], c[_NP:]], axis=1)
        e = jnp.dot(ea, _bd2(we1[2 * _H:]), preferred_element_type=_F32)
        pre = (e.reshape(_N, _NP, _HP)
               + ap.reshape(_N, 1, _HP)
               + cp.reshape(1, _NP, _HP))
        m1 = jnp.maximum(pre, 0.0).reshape(_EP, _HP).astype(_BF)
        be2 = be2_ref[...]
        m2 = jnp.maximum(
            jnp.dot(m1, _bd2(we2_ref[...]), preferred_element_type=_F32)
            + jnp.concatenate([be2, be2], axis=1), 0.0)
        ms = m2.reshape(_N, _NP, _HP).sum(axis=1)
        mi = ms[:, :_H] + ms[:, _H:]
        wn1 = wn1_ref[...]
        u = jnp.maximum(
            jnp.dot(hb, wn1[:_H].astype(_BF), preferred_element_type=_F32)
            + jnp.dot(mi.astype(_BF), wn1[_H:].astype(_BF), preferred_element_type=_F32)
            + bn1_ref[...], 0.0)
        h = (h + jnp.dot(u.astype(_BF), wn2_ref[...].astype(_BF),
                         preferred_element_type=_F32)
             + bn2_ref[...])

    h_ref[0] = h


def _out_body(hf_ref, w1_ref, b1_ref, w2_ref, b2_ref, o_ref):
    h1 = jnp.maximum(
        jnp.dot(hf_ref[...], w1_ref[...].astype(_BF), preferred_element_type=_F32)
        + b1_ref[...], 0.0)
    o_ref[...] = (jnp.dot(h1.astype(_BF), w2_ref[...].astype(_BF),
                          preferred_element_type=_F32)
                  + b2_ref[...])


def kernel(x, edge_attr, key_unused, params):
    p = params
    wl = []
    for l in range(_NL):
        wl += [p['l%d_We1' % l], p['l%d_be1' % l].reshape(1, _H),
               p['l%d_We2' % l], p['l%d_be2' % l].reshape(1, _H),
               p['l%d_Wn1' % l], p['l%d_bn1' % l].reshape(1, _H),
               p['l%d_Wn2' % l], p['l%d_bn2' % l].reshape(1, _H)]
    eap = (edge_attr.astype(_BF)
           .reshape(_B, _N, 2, _NP, _DE)
           .transpose(0, 1, 3, 2, 4)
           .reshape(_B, _EP, 2 * _DE))

    _cw = lambda b: (0, 0)
    h = pl.pallas_call(
        _gnn_body,
        grid=(_B,),
        in_specs=[
            pl.BlockSpec((1, _N, _DIN), lambda b: (b, 0, 0)),
            pl.BlockSpec((1, _EP, 2 * _DE), lambda b: (b, 0, 0)),
            pl.BlockSpec((_DIN, _H), _cw),
            pl.BlockSpec((1, _H), _cw),
            pl.BlockSpec((_H, _H), _cw),
            pl.BlockSpec((1, _H), _cw),
        ] + [pl.BlockSpec(w.shape, _cw) for w in wl],
        out_specs=pl.BlockSpec((1, _N, _H), lambda b: (b, 0, 0)),
        out_shape=jax.ShapeDtypeStruct((_B, _N, _H), _F32),
        compiler_params=pltpu.CompilerParams(
            dimension_semantics=("arbitrary",)),
    )(x, eap, p['emb_in_W1'], p['emb_in_b1'].reshape(1, _H),
      p['emb_in_W2'], p['emb_in_b2'].reshape(1, _H), *wl)

    hf = h.reshape(_B, _N * _H).astype(_BF)
    out = pl.pallas_call(
        _out_body,
        out_shape=jax.ShapeDtypeStruct((_B, _OUT), _F32),
    )(hf, p['emb_out_W1'], p['emb_out_b1'].reshape(1, _H),
      p['emb_out_W2'], p['emb_out_b2'].reshape(1, _OUT))
    return out
